# baseline (speedup 1.0000x reference)
import jax
import jax.numpy as jnp
from jax import lax
from jax.experimental import pallas as pl
from jax.experimental.pallas import tpu as pltpu
from jax.experimental.pallas import tpu_sc as plsc

H = 128
L = 16
NC = 2
NS = 16
NW = NC * NS
CHUNK = 256
BN_EPS = 1e-3


_SC_PARAMS = pltpu.CompilerParams(use_tc_tiling_on_sc=False)
_MESH = dict(core_axis_name="c", subcore_axis_name="s")


def _make_sc_segment_sum(n_pad, k0, k1):
    rows_per_tile = n_pad // NS
    for kc in (k0, k1):
        assert kc >= 4 and (kc - 2) % 3 == 0

    scratch = (
        [pltpu.VMEM((CHUNK,), jnp.int32) for _ in range(6)]
        + [pltpu.VMEM((CHUNK, H), jnp.float32)]
        + [pltpu.VMEM_SHARED((n_pad, H), jnp.float32)]
        + [pltpu.SemaphoreType.DMA for _ in range(4)]
    )

    def body(h, src_i, dst_i, zeros_s, s_out,
             is0, is1, is2, id0, id1, id2, rows, s_sh,
             i0, i1, i2, gsem):
        isrcb = [is0, is1, is2]
        idstb = [id0, id1, id2]
        isem = [i0, i1, i2]

        c = lax.axis_index("c")
        s = lax.axis_index("s")
        row0 = s * rows_per_tile

        pltpu.sync_copy(zeros_s.at[pl.ds(row0, rows_per_tile)],
                        s_sh.at[pl.ds(row0, rows_per_tile)])
        plsc.subcore_barrier()

        def pipeline(kk, base):
            def idx_start(i, ib):
                pltpu.async_copy(src_i.at[base + i], isrcb[ib], isem[ib])
                pltpu.async_copy(dst_i.at[base + i], idstb[ib], isem[ib])

            def idx_wait(i, ib):
                pltpu.make_async_copy(src_i.at[base + i], isrcb[ib],
                                      isem[ib]).wait()
                pltpu.make_async_copy(dst_i.at[base + i], idstb[ib],
                                      isem[ib]).wait()

            def step(i, ib, do_idx):
                if do_idx:
                    idx_start(i + 2, (ib + 2) % 3)
                idx_wait(i, ib)
                pltpu.async_copy(h.at[isrcb[ib]], rows, gsem).wait()
                pltpu.sync_copy(rows, s_sh.at[idstb[ib]], add=True)

            idx_start(0, 0)
            idx_start(1, 1)

            def mid(o, carry):
                for t in range(3):
                    step(o * 3 + t, t, True)
                return carry

            lax.fori_loop(0, (kk - 2) // 3, mid, 0)
            for i in range(kk - 2, kk):
                step(i, i % 3, False)

        if k0 == k1:
            pipeline(k0, (c * NS + s) * k0)
        else:
            @pl.when(c == 0)
            def _():
                pipeline(k0, s * k0)

            @pl.when(c == 1)
            def _():
                pipeline(k1, NS * k0 + s * k1)

        plsc.subcore_barrier()
        pltpu.sync_copy(s_sh.at[pl.ds(row0, rows_per_tile)],
                        s_out.at[c, pl.ds(row0, rows_per_tile)])

    return pl.kernel(
        body, out_type=jax.ShapeDtypeStruct((NC, n_pad, H), jnp.float32),
        mesh=plsc.VectorSubcoreMesh(**_MESH),
        scratch_types=tuple(scratch), compiler_params=_SC_PARAMS)


def _make_sc_degree(n_pad, kk):
    rows_per_tile = n_pad // NS

    scratch = (
        pltpu.VMEM((kk, CHUNK), jnp.int32),
        pltpu.VMEM((CHUNK, L), jnp.float32),
        pltpu.VMEM_SHARED((n_pad, L), jnp.float32),
        pltpu.SemaphoreType.DMA,
    )

    def body(dst_i, zeros_d, ones, deg_out, idst, ones_v, deg_sh, dsem):
        c = lax.axis_index("c")
        s = lax.axis_index("s")
        wid = s * NC + c
        row0 = s * rows_per_tile

        pltpu.sync_copy(dst_i.at[pl.ds(wid * kk, kk)], idst)
        pltpu.sync_copy(ones, ones_v)
        pltpu.sync_copy(zeros_d.at[pl.ds(row0, rows_per_tile)],
                        deg_sh.at[pl.ds(row0, rows_per_tile)])
        plsc.subcore_barrier()

        def blk(o, carry):
            for t in range(4):
                pltpu.async_copy(ones_v, deg_sh.at[idst.at[o * 4 + t]],
                                 dsem, add=True)
            for t in range(4):
                pltpu.make_async_copy(ones_v, deg_sh.at[idst.at[o * 4 + t]],
                                      dsem).wait()
            return carry

        lax.fori_loop(0, kk // 4, blk, 0)
        for i in range(kk - kk % 4, kk):
            pltpu.sync_copy(ones_v, deg_sh.at[idst.at[i]], add=True)

        plsc.subcore_barrier()
        pltpu.sync_copy(deg_sh.at[pl.ds(row0, rows_per_tile)],
                        deg_out.at[c, pl.ds(row0, rows_per_tile)])

    return pl.kernel(
        body, out_type=jax.ShapeDtypeStruct((NC, n_pad, L), jnp.float32),
        mesh=plsc.VectorSubcoreMesh(**_MESH),
        scratch_types=scratch, compiler_params=_SC_PARAMS)


def _emb_body(x_ref, w_ref, b_ref, o_ref):
    o_ref[...] = jnp.dot(x_ref[...], w_ref[...],
                         preferred_element_type=jnp.float32,
                         precision=lax.Precision.HIGHEST) + b_ref[...]


def _embed(x, w, b, blk):
    n = x.shape[0]
    return pl.pallas_call(
        _emb_body,
        grid=(n // blk,),
        in_specs=[
            pl.BlockSpec((blk, H), lambda i: (i, 0)),
            pl.BlockSpec((H, H), lambda i: (0, 0)),
            pl.BlockSpec((1, H), lambda i: (0, 0)),
        ],
        out_specs=pl.BlockSpec((blk, H), lambda i: (i, 0)),
        out_shape=jax.ShapeDtypeStruct((n, H), jnp.float32),
    )(x, w, b.reshape(1, H))


def _layer_body(h_ref, s_ref, deg_ref, wt_ref, wb_ref, p_ref, o_ref):
    h = h_ref[...]
    s_sum = s_ref[0] + s_ref[1]
    deg = deg_ref[0, :, 0:1] + deg_ref[1, :, 0:1]
    b = p_ref[0:1, :]
    gamma = p_ref[1:2, :]
    beta = p_ref[2:3, :]
    mean = p_ref[3:4, :]
    var = p_ref[4:5, :]
    hw = jnp.dot(h, wb_ref[...], preferred_element_type=jnp.float32,
                 precision=lax.Precision.HIGHEST)
    agg = jnp.dot(s_sum, wt_ref[...], preferred_element_type=jnp.float32,
                  precision=lax.Precision.HIGHEST) + deg * (hw + b)
    z = jax.nn.sigmoid(agg) + jax.nn.softplus(h)
    z = (z - mean) / jnp.sqrt(var + BN_EPS) * gamma + beta
    o_ref[...] = jnp.maximum(z, 0.0)


def _layer_dense(h, s_partial, deg_partial, w, params, blk):
    n = h.shape[0]
    return pl.pallas_call(
        _layer_body,
        grid=(n // blk,),
        in_specs=[
            pl.BlockSpec((blk, H), lambda i: (i, 0)),
            pl.BlockSpec((NC, blk, H), lambda i: (0, i, 0)),
            pl.BlockSpec((NC, blk, L), lambda i: (0, i, 0)),
            pl.BlockSpec((H, H), lambda i: (0, 0)),
            pl.BlockSpec((H, H), lambda i: (0, 0)),
            pl.BlockSpec((5, H), lambda i: (0, 0)),
        ],
        out_specs=pl.BlockSpec((blk, H), lambda i: (i, 0)),
        out_shape=jax.ShapeDtypeStruct((n, H), jnp.float32),
    )(h, s_partial, deg_partial, w[:H], w[H:], params)


def kernel(node_feat, edge_index, W_emb, b_emb,
           w0, b0, gamma0, beta0, mean0, var0,
           w1, b1, gamma1, beta1, mean1, var1):
    n = node_feat.shape[0]
    e = edge_index.shape[1]
    n_pad = ((n + 1 + NS * 8 - 1) // (NS * 8)) * (NS * 8)
    p_tot = (e + NS * CHUNK - 1) // (NS * CHUNK)
    while p_tot % 2 or p_tot % 3 != 1:
        p_tot += 1
    k1 = max(5, int(p_tot * 0.2375))
    k1 = (k1 // 3) * 3 + 2
    k0 = p_tot - k1
    totc = NS * p_tot
    e_pad = totc * CHUNK

    src = edge_index[0].astype(jnp.int32)
    dst = edge_index[1].astype(jnp.int32)
    src_i = jnp.zeros((e_pad,), jnp.int32).at[:e].set(src).reshape(totc, CHUNK)
    dst_i = jnp.full((e_pad,), n, jnp.int32).at[:e].set(dst).reshape(totc, CHUNK)

    zeros_s = jnp.zeros((n_pad, H), jnp.float32)
    zeros_d = jnp.zeros((n_pad, L), jnp.float32)
    ones = jnp.ones((CHUNK, L), jnp.float32)

    sc_seg = _make_sc_segment_sum(n_pad, k0, k1)
    sc_deg = _make_sc_degree(n_pad, totc // NW)

    blk = 2000
    h = _embed(node_feat, W_emb, b_emb, blk)
    deg_p = sc_deg(dst_i, zeros_d, ones)

    s_p = sc_seg(h, src_i, dst_i, zeros_s)
    params0 = jnp.stack([b0, gamma0, beta0, mean0, var0])
    h = _layer_dense(h, s_p, deg_p, w0, params0, blk)

    s_p = sc_seg(h, src_i, dst_i, zeros_s)
    params1 = jnp.stack([b1, gamma1, beta1, mean1, var1])
    h = _layer_dense(h, s_p, deg_p, w1, params1, blk)

    return h


# speedup vs baseline: 1.3709x; 1.3709x over previous
import jax
import jax.numpy as jnp
from jax import lax
from jax.experimental import pallas as pl
from jax.experimental.pallas import tpu as pltpu
from jax.experimental.pallas import tpu_sc as plsc

H = 128
L = 16
NC = 2
NS = 16
NW = NC * NS
CHUNK = 128
BN_EPS = 1e-3


_SC_PARAMS = pltpu.CompilerParams(use_tc_tiling_on_sc=False)
_MESH = dict(core_axis_name="c", subcore_axis_name="s")


def _make_sc_segment_sum(n_pad, k0, k1):
    rows_per_tile = n_pad // NS
    for kc in (k0, k1):
        assert kc >= 4 and (kc - 2) % 3 == 0

    scratch = (
        [pltpu.VMEM((CHUNK,), jnp.int32) for _ in range(6)]
        + [pltpu.VMEM((CHUNK, H), jnp.float32)]
        + [pltpu.VMEM_SHARED((n_pad, H), jnp.float32)]
        + [pltpu.SemaphoreType.DMA for _ in range(4)]
    )

    def body(h, src_i, dst_i, zeros_s, s_out,
             is0, is1, is2, id0, id1, id2, rows, s_sh,
             i0, i1, i2, gsem):
        isrcb = [is0, is1, is2]
        idstb = [id0, id1, id2]
        isem = [i0, i1, i2]

        c = lax.axis_index("c")
        s = lax.axis_index("s")
        row0 = s * rows_per_tile

        pltpu.sync_copy(zeros_s.at[pl.ds(row0, rows_per_tile)],
                        s_sh.at[pl.ds(row0, rows_per_tile)])
        plsc.subcore_barrier()

        def pipeline(kk, base):
            def idx_start(i, ib):
                pltpu.async_copy(src_i.at[base + i], isrcb[ib], isem[ib])
                pltpu.async_copy(dst_i.at[base + i], idstb[ib], isem[ib])

            def idx_wait(i, ib):
                pltpu.make_async_copy(src_i.at[base + i], isrcb[ib],
                                      isem[ib]).wait()
                pltpu.make_async_copy(dst_i.at[base + i], idstb[ib],
                                      isem[ib]).wait()

            def step(i, ib, do_idx):
                if do_idx:
                    idx_start(i + 2, (ib + 2) % 3)
                idx_wait(i, ib)
                pltpu.async_copy(h.at[isrcb[ib]], rows, gsem).wait()
                pltpu.sync_copy(rows, s_sh.at[idstb[ib]], add=True)

            idx_start(0, 0)
            idx_start(1, 1)

            def mid(o, carry):
                for t in range(3):
                    step(o * 3 + t, t, True)
                return carry

            lax.fori_loop(0, (kk - 2) // 3, mid, 0)
            for i in range(kk - 2, kk):
                step(i, i % 3, False)

        if k0 == k1:
            pipeline(k0, (c * NS + s) * k0)
        else:
            @pl.when(c == 0)
            def _():
                pipeline(k0, s * k0)

            @pl.when(c == 1)
            def _():
                pipeline(k1, NS * k0 + s * k1)

        plsc.subcore_barrier()
        pltpu.sync_copy(s_sh.at[pl.ds(row0, rows_per_tile)],
                        s_out.at[c, pl.ds(row0, rows_per_tile)])

    return pl.kernel(
        body, out_type=jax.ShapeDtypeStruct((NC, n_pad, H), jnp.float32),
        mesh=plsc.VectorSubcoreMesh(**_MESH),
        scratch_types=tuple(scratch), compiler_params=_SC_PARAMS)


def _make_sc_degree(n_pad, kk):
    rows_per_tile = n_pad // NS

    scratch = (
        pltpu.VMEM((kk, CHUNK), jnp.int32),
        pltpu.VMEM((CHUNK, L), jnp.float32),
        pltpu.VMEM_SHARED((n_pad, L), jnp.float32),
        pltpu.SemaphoreType.DMA,
    )

    def body(dst_i, zeros_d, ones, deg_out, idst, ones_v, deg_sh, dsem):
        c = lax.axis_index("c")
        s = lax.axis_index("s")
        wid = s * NC + c
        row0 = s * rows_per_tile

        pltpu.sync_copy(dst_i.at[pl.ds(wid * kk, kk)], idst)
        pltpu.sync_copy(ones, ones_v)
        pltpu.sync_copy(zeros_d.at[pl.ds(row0, rows_per_tile)],
                        deg_sh.at[pl.ds(row0, rows_per_tile)])
        plsc.subcore_barrier()

        def blk(o, carry):
            for t in range(4):
                pltpu.async_copy(ones_v, deg_sh.at[idst.at[o * 4 + t]],
                                 dsem, add=True)
            for t in range(4):
                pltpu.make_async_copy(ones_v, deg_sh.at[idst.at[o * 4 + t]],
                                      dsem).wait()
            return carry

        lax.fori_loop(0, kk // 4, blk, 0)
        for i in range(kk - kk % 4, kk):
            pltpu.sync_copy(ones_v, deg_sh.at[idst.at[i]], add=True)

        plsc.subcore_barrier()
        pltpu.sync_copy(deg_sh.at[pl.ds(row0, rows_per_tile)],
                        deg_out.at[c, pl.ds(row0, rows_per_tile)])

    return pl.kernel(
        body, out_type=jax.ShapeDtypeStruct((NC, n_pad, L), jnp.float32),
        mesh=plsc.VectorSubcoreMesh(**_MESH),
        scratch_types=scratch, compiler_params=_SC_PARAMS)


def _emb_body(x_ref, w_ref, b_ref, o_ref):
    o_ref[...] = jnp.dot(x_ref[...], w_ref[...],
                         preferred_element_type=jnp.float32,
                         precision=lax.Precision.HIGHEST) + b_ref[...]


def _embed(x, w, b, blk):
    n = x.shape[0]
    return pl.pallas_call(
        _emb_body,
        grid=(n // blk,),
        in_specs=[
            pl.BlockSpec((blk, H), lambda i: (i, 0)),
            pl.BlockSpec((H, H), lambda i: (0, 0)),
            pl.BlockSpec((1, H), lambda i: (0, 0)),
        ],
        out_specs=pl.BlockSpec((blk, H), lambda i: (i, 0)),
        out_shape=jax.ShapeDtypeStruct((n, H), jnp.float32),
    )(x, w, b.reshape(1, H))


def _layer_body(h_ref, s_ref, deg_ref, wt_ref, wb_ref, p_ref, o_ref):
    h = h_ref[...]
    s_sum = s_ref[0] + s_ref[1]
    deg = deg_ref[0, :, 0:1] + deg_ref[1, :, 0:1]
    b = p_ref[0:1, :]
    gamma = p_ref[1:2, :]
    beta = p_ref[2:3, :]
    mean = p_ref[3:4, :]
    var = p_ref[4:5, :]
    hw = jnp.dot(h, wb_ref[...], preferred_element_type=jnp.float32,
                 precision=lax.Precision.HIGHEST)
    agg = jnp.dot(s_sum, wt_ref[...], preferred_element_type=jnp.float32,
                  precision=lax.Precision.HIGHEST) + deg * (hw + b)
    z = jax.nn.sigmoid(agg) + jax.nn.softplus(h)
    z = (z - mean) / jnp.sqrt(var + BN_EPS) * gamma + beta
    o_ref[...] = jnp.maximum(z, 0.0)


def _layer_dense(h, s_partial, deg_partial, w, params, blk):
    n = h.shape[0]
    return pl.pallas_call(
        _layer_body,
        grid=(n // blk,),
        in_specs=[
            pl.BlockSpec((blk, H), lambda i: (i, 0)),
            pl.BlockSpec((NC, blk, H), lambda i: (0, i, 0)),
            pl.BlockSpec((NC, blk, L), lambda i: (0, i, 0)),
            pl.BlockSpec((H, H), lambda i: (0, 0)),
            pl.BlockSpec((H, H), lambda i: (0, 0)),
            pl.BlockSpec((5, H), lambda i: (0, 0)),
        ],
        out_specs=pl.BlockSpec((blk, H), lambda i: (i, 0)),
        out_shape=jax.ShapeDtypeStruct((n, H), jnp.float32),
    )(h, s_partial, deg_partial, w[:H], w[H:], params)


def kernel(node_feat, edge_index, W_emb, b_emb,
           w0, b0, gamma0, beta0, mean0, var0,
           w1, b1, gamma1, beta1, mean1, var1):
    n = node_feat.shape[0]
    e = edge_index.shape[1]
    n_pad = ((n + 1 + NS * 8 - 1) // (NS * 8)) * (NS * 8)
    p_tot = (e + NS * CHUNK - 1) // (NS * CHUNK)
    while p_tot % 2 or p_tot % 3 != 1:
        p_tot += 1
    k1 = max(5, int(p_tot * 0.5))
    k1 = (k1 // 3) * 3 + 2
    k0 = p_tot - k1
    totc = NS * p_tot
    e_pad = totc * CHUNK

    src = edge_index[0].astype(jnp.int32)
    dst = edge_index[1].astype(jnp.int32)
    src_i = jnp.zeros((e_pad,), jnp.int32).at[:e].set(src).reshape(totc, CHUNK)
    dst_i = jnp.full((e_pad,), n, jnp.int32).at[:e].set(dst).reshape(totc, CHUNK)

    zeros_s = jnp.zeros((n_pad, H), jnp.float32)
    zeros_d = jnp.zeros((n_pad, L), jnp.float32)
    ones = jnp.ones((CHUNK, L), jnp.float32)

    sc_seg = _make_sc_segment_sum(n_pad, k0, k1)
    sc_deg = _make_sc_degree(n_pad, totc // NW)

    blk = 2000
    h = _embed(node_feat, W_emb, b_emb, blk)
    deg_p = sc_deg(dst_i, zeros_d, ones)

    s_p = sc_seg(h, src_i, dst_i, zeros_s)
    params0 = jnp.stack([b0, gamma0, beta0, mean0, var0])
    h = _layer_dense(h, s_p, deg_p, w0, params0, blk)

    s_p = sc_seg(h, src_i, dst_i, zeros_s)
    params1 = jnp.stack([b1, gamma1, beta1, mean1, var1])
    h = _layer_dense(h, s_p, deg_p, w1, params1, blk)

    return h


# speedup vs baseline: 1.5251x; 1.1124x over previous
import jax
import jax.numpy as jnp
from jax import lax
from jax.experimental import pallas as pl
from jax.experimental.pallas import tpu as pltpu
from jax.experimental.pallas import tpu_sc as plsc

H = 128
L = 16
NC = 2
NS = 16
NW = NC * NS
CHUNK = 128
BN_EPS = 1e-3


_SC_PARAMS = pltpu.CompilerParams(use_tc_tiling_on_sc=False)
_MESH = dict(core_axis_name="c", subcore_axis_name="s")


def _make_sc_segment_sum(n_pad, k0, k1):
    rows_per_tile = n_pad // NS
    for kc in (k0, k1):
        assert kc >= 4 and (kc - 2) % 3 == 0

    scratch = (
        [pltpu.VMEM((CHUNK,), jnp.int32) for _ in range(6)]
        + [pltpu.VMEM((CHUNK, H), jnp.float32)]
        + [pltpu.VMEM_SHARED((n_pad, H), jnp.float32)]
        + [pltpu.SemaphoreType.DMA for _ in range(4)]
    )

    def body(h, src_i, dst_i, zeros_s, s_out,
             is0, is1, is2, id0, id1, id2, rows, s_sh,
             i0, i1, i2, gsem):
        isrcb = [is0, is1, is2]
        idstb = [id0, id1, id2]
        isem = [i0, i1, i2]

        c = lax.axis_index("c")
        s = lax.axis_index("s")
        row0 = s * rows_per_tile

        pltpu.sync_copy(zeros_s.at[pl.ds(row0, rows_per_tile)],
                        s_sh.at[pl.ds(row0, rows_per_tile)])
        plsc.subcore_barrier()

        def pipeline(kk, base):
            def idx_start(i, ib):
                pltpu.async_copy(src_i.at[base + i], isrcb[ib], isem[ib])
                pltpu.async_copy(dst_i.at[base + i], idstb[ib], isem[ib])

            def idx_wait(i, ib):
                pltpu.make_async_copy(src_i.at[base + i], isrcb[ib],
                                      isem[ib]).wait()
                pltpu.make_async_copy(dst_i.at[base + i], idstb[ib],
                                      isem[ib]).wait()

            def step(i, ib, do_idx):
                if do_idx:
                    idx_start(i + 2, (ib + 2) % 3)
                idx_wait(i, ib)
                pltpu.async_copy(h.at[isrcb[ib]], rows, gsem).wait()
                pltpu.sync_copy(rows, s_sh.at[idstb[ib]], add=True)

            idx_start(0, 0)
            idx_start(1, 1)

            def mid(o, carry):
                for t in range(3):
                    step(o * 3 + t, t, True)
                return carry

            lax.fori_loop(0, (kk - 2) // 3, mid, 0)
            for i in range(kk - 2, kk):
                step(i, i % 3, False)

        if k0 == k1:
            pipeline(k0, (c * NS + s) * k0)
        else:
            @pl.when(c == 0)
            def _():
                pipeline(k0, s * k0)

            @pl.when(c == 1)
            def _():
                pipeline(k1, NS * k0 + s * k1)

        plsc.subcore_barrier()
        pltpu.sync_copy(s_sh.at[pl.ds(row0, rows_per_tile)],
                        s_out.at[c, pl.ds(row0, rows_per_tile)])

    return pl.kernel(
        body, out_type=jax.ShapeDtypeStruct((NC, n_pad, H), jnp.float32),
        mesh=plsc.VectorSubcoreMesh(**_MESH),
        scratch_types=tuple(scratch), compiler_params=_SC_PARAMS)


def _make_sc_degree(n_pad, kk):
    rows_per_tile = n_pad // NS

    scratch = (
        pltpu.VMEM((kk, CHUNK), jnp.int32),
        pltpu.VMEM((CHUNK, L), jnp.float32),
        pltpu.VMEM_SHARED((n_pad, L), jnp.float32),
        pltpu.SemaphoreType.DMA,
    )

    def body(dst_i, zeros_d, ones, deg_out, idst, ones_v, deg_sh, dsem):
        c = lax.axis_index("c")
        s = lax.axis_index("s")
        wid = s * NC + c
        row0 = s * rows_per_tile

        pltpu.sync_copy(dst_i.at[pl.ds(wid * kk, kk)], idst)
        pltpu.sync_copy(ones, ones_v)
        pltpu.sync_copy(zeros_d.at[pl.ds(row0, rows_per_tile)],
                        deg_sh.at[pl.ds(row0, rows_per_tile)])
        plsc.subcore_barrier()

        def blk(o, carry):
            for t in range(4):
                pltpu.async_copy(ones_v, deg_sh.at[idst.at[o * 4 + t]],
                                 dsem, add=True)
            for t in range(4):
                pltpu.make_async_copy(ones_v, deg_sh.at[idst.at[o * 4 + t]],
                                      dsem).wait()
            return carry

        lax.fori_loop(0, kk // 4, blk, 0)
        for i in range(kk - kk % 4, kk):
            pltpu.sync_copy(ones_v, deg_sh.at[idst.at[i]], add=True)

        plsc.subcore_barrier()
        pltpu.sync_copy(deg_sh.at[pl.ds(row0, rows_per_tile)],
                        deg_out.at[c, pl.ds(row0, rows_per_tile)])

    return pl.kernel(
        body, out_type=jax.ShapeDtypeStruct((NC, n_pad, L), jnp.float32),
        mesh=plsc.VectorSubcoreMesh(**_MESH),
        scratch_types=scratch, compiler_params=_SC_PARAMS)


def _emb_body(x_ref, w_ref, b_ref, o_ref):
    o_ref[...] = jnp.dot(x_ref[...], w_ref[...],
                         preferred_element_type=jnp.float32,
                         precision=lax.Precision.HIGHEST) + b_ref[...]


def _embed(x, w, b, blk):
    n = x.shape[0]
    return pl.pallas_call(
        _emb_body,
        grid=(n // blk,),
        in_specs=[
            pl.BlockSpec((blk, H), lambda i: (i, 0)),
            pl.BlockSpec((H, H), lambda i: (0, 0)),
            pl.BlockSpec((1, H), lambda i: (0, 0)),
        ],
        out_specs=pl.BlockSpec((blk, H), lambda i: (i, 0)),
        out_shape=jax.ShapeDtypeStruct((n, H), jnp.float32),
    )(x, w, b.reshape(1, H))


def _layer_body(h_ref, s_ref, deg_ref, wt_ref, wb_ref, p_ref, o_ref):
    h = h_ref[...]
    s_sum = s_ref[0] + s_ref[1]
    deg = deg_ref[0, :, 0:1] + deg_ref[1, :, 0:1]
    b = p_ref[0:1, :]
    gamma = p_ref[1:2, :]
    beta = p_ref[2:3, :]
    mean = p_ref[3:4, :]
    var = p_ref[4:5, :]
    hw = jnp.dot(h, wb_ref[...], preferred_element_type=jnp.float32,
                 precision=lax.Precision.HIGHEST)
    agg = jnp.dot(s_sum, wt_ref[...], preferred_element_type=jnp.float32,
                  precision=lax.Precision.HIGHEST) + deg * (hw + b)
    z = jax.nn.sigmoid(agg) + jax.nn.softplus(h)
    z = (z - mean) / jnp.sqrt(var + BN_EPS) * gamma + beta
    o_ref[...] = jnp.maximum(z, 0.0)


def _layer_dense(h, s_partial, deg_partial, w, params, blk):
    n = h.shape[0]
    return pl.pallas_call(
        _layer_body,
        grid=(n // blk,),
        in_specs=[
            pl.BlockSpec((blk, H), lambda i: (i, 0)),
            pl.BlockSpec((NC, blk, H), lambda i: (0, i, 0)),
            pl.BlockSpec((NC, blk, L), lambda i: (0, i, 0)),
            pl.BlockSpec((H, H), lambda i: (0, 0)),
            pl.BlockSpec((H, H), lambda i: (0, 0)),
            pl.BlockSpec((5, H), lambda i: (0, 0)),
        ],
        out_specs=pl.BlockSpec((blk, H), lambda i: (i, 0)),
        out_shape=jax.ShapeDtypeStruct((n, H), jnp.float32),
    )(h, s_partial, deg_partial, w[:H], w[H:], params)


def kernel(node_feat, edge_index, W_emb, b_emb,
           w0, b0, gamma0, beta0, mean0, var0,
           w1, b1, gamma1, beta1, mean1, var1):
    n = node_feat.shape[0]
    e = edge_index.shape[1]
    n_pad = ((n + 1 + NS * 8 - 1) // (NS * 8)) * (NS * 8)
    p_tot = (e + NS * CHUNK - 1) // (NS * CHUNK)
    while p_tot % 2 or p_tot % 3 != 1:
        p_tot += 1
    k1 = max(5, int(p_tot * 0.35))
    k1 = (k1 // 3) * 3 + 2
    k0 = p_tot - k1
    totc = NS * p_tot
    e_pad = totc * CHUNK

    src = edge_index[0].astype(jnp.int32)
    dst = edge_index[1].astype(jnp.int32)
    src_i = jnp.zeros((e_pad,), jnp.int32).at[:e].set(src).reshape(totc, CHUNK)
    dst_i = jnp.full((e_pad,), n, jnp.int32).at[:e].set(dst).reshape(totc, CHUNK)

    zeros_s = jnp.zeros((n_pad, H), jnp.float32)
    zeros_d = jnp.zeros((n_pad, L), jnp.float32)
    ones = jnp.ones((CHUNK, L), jnp.float32)

    sc_seg = _make_sc_segment_sum(n_pad, k0, k1)
    sc_deg = _make_sc_degree(n_pad, totc // NW)

    blk = 2000
    h = _embed(node_feat, W_emb, b_emb, blk)
    deg_p = sc_deg(dst_i, zeros_d, ones)

    s_p = sc_seg(h, src_i, dst_i, zeros_s)
    params0 = jnp.stack([b0, gamma0, beta0, mean0, var0])
    h = _layer_dense(h, s_p, deg_p, w0, params0, blk)

    s_p = sc_seg(h, src_i, dst_i, zeros_s)
    params1 = jnp.stack([b1, gamma1, beta1, mean1, var1])
    h = _layer_dense(h, s_p, deg_p, w1, params1, blk)

    return h


# speedup vs baseline: 1.8373x; 1.2047x over previous
import jax
import jax.numpy as jnp
from jax import lax
from jax.experimental import pallas as pl
from jax.experimental.pallas import tpu as pltpu
from jax.experimental.pallas import tpu_sc as plsc

H = 128
L = 16
NC = 2
NS = 16
NW = NC * NS
CHUNK = 128
BN_EPS = 1e-3


_SC_PARAMS = pltpu.CompilerParams(use_tc_tiling_on_sc=False)
_MESH = dict(core_axis_name="c", subcore_axis_name="s")


def _make_sc_segment_sum(n_pad, k0, k1):
    rows_per_tile = n_pad // NS
    for kc in (k0, k1):
        assert kc >= 4 and (kc - 2) % 3 == 0

    scratch = (
        [pltpu.VMEM((CHUNK,), jnp.int32) for _ in range(6)]
        + [pltpu.VMEM((CHUNK, H), jnp.float32)]
        + [pltpu.VMEM_SHARED((n_pad, H), jnp.float32)]
        + [pltpu.SemaphoreType.DMA for _ in range(4)]
    )

    def body(h, src_i, dst_i, zeros_s, s_out,
             is0, is1, is2, id0, id1, id2, rows, s_sh,
             i0, i1, i2, gsem):
        isrcb = [is0, is1, is2]
        idstb = [id0, id1, id2]
        isem = [i0, i1, i2]

        c = lax.axis_index("c")
        s = lax.axis_index("s")
        row0 = s * rows_per_tile

        pltpu.sync_copy(zeros_s.at[pl.ds(row0, rows_per_tile)],
                        s_sh.at[pl.ds(row0, rows_per_tile)])
        plsc.subcore_barrier()

        def pipeline(kk, base):
            def idx_start(i, ib):
                pltpu.async_copy(src_i.at[base + i], isrcb[ib], isem[ib])
                pltpu.async_copy(dst_i.at[base + i], idstb[ib], isem[ib])

            def idx_wait(i, ib):
                pltpu.make_async_copy(src_i.at[base + i], isrcb[ib],
                                      isem[ib]).wait()
                pltpu.make_async_copy(dst_i.at[base + i], idstb[ib],
                                      isem[ib]).wait()

            def step(i, ib, do_idx):
                if do_idx:
                    idx_start(i + 2, (ib + 2) % 3)
                idx_wait(i, ib)
                pltpu.async_copy(h.at[isrcb[ib]], rows, gsem).wait()
                pltpu.sync_copy(rows, s_sh.at[idstb[ib]], add=True)

            idx_start(0, 0)
            idx_start(1, 1)

            def mid(o, carry):
                for t in range(3):
                    step(o * 3 + t, t, True)
                return carry

            lax.fori_loop(0, (kk - 2) // 3, mid, 0)
            for i in range(kk - 2, kk):
                step(i, i % 3, False)

        if k0 == k1:
            pipeline(k0, (c * NS + s) * k0)
        else:
            @pl.when(c == 0)
            def _():
                pipeline(k0, s * k0)

            @pl.when(c == 1)
            def _():
                pipeline(k1, NS * k0 + s * k1)

        plsc.subcore_barrier()
        pltpu.sync_copy(s_sh.at[pl.ds(row0, rows_per_tile)],
                        s_out.at[c, pl.ds(row0, rows_per_tile)])

    return pl.kernel(
        body, out_type=jax.ShapeDtypeStruct((NC, n_pad, H), jnp.float32),
        mesh=plsc.VectorSubcoreMesh(**_MESH),
        scratch_types=tuple(scratch), compiler_params=_SC_PARAMS)


def _make_sc_degree(n_pad, kk):
    rows_per_tile = n_pad // NS

    scratch = (
        pltpu.VMEM((kk, CHUNK), jnp.int32),
        pltpu.VMEM((CHUNK, L), jnp.float32),
        pltpu.VMEM_SHARED((n_pad, L), jnp.float32),
        pltpu.SemaphoreType.DMA,
    )

    def body(dst_i, zeros_d, ones, deg_out, idst, ones_v, deg_sh, dsem):
        c = lax.axis_index("c")
        s = lax.axis_index("s")
        wid = s * NC + c
        row0 = s * rows_per_tile

        pltpu.sync_copy(dst_i.at[pl.ds(wid * kk, kk)], idst)
        pltpu.sync_copy(ones, ones_v)
        pltpu.sync_copy(zeros_d.at[pl.ds(row0, rows_per_tile)],
                        deg_sh.at[pl.ds(row0, rows_per_tile)])
        plsc.subcore_barrier()

        def blk(o, carry):
            for t in range(4):
                pltpu.async_copy(ones_v, deg_sh.at[idst.at[o * 4 + t]],
                                 dsem, add=True)
            for t in range(4):
                pltpu.make_async_copy(ones_v, deg_sh.at[idst.at[o * 4 + t]],
                                      dsem).wait()
            return carry

        lax.fori_loop(0, kk // 4, blk, 0)
        for i in range(kk - kk % 4, kk):
            pltpu.sync_copy(ones_v, deg_sh.at[idst.at[i]], add=True)

        plsc.subcore_barrier()
        pltpu.sync_copy(deg_sh.at[pl.ds(row0, rows_per_tile)],
                        deg_out.at[c, pl.ds(row0, rows_per_tile)])

    return pl.kernel(
        body, out_type=jax.ShapeDtypeStruct((NC, n_pad, L), jnp.float32),
        mesh=plsc.VectorSubcoreMesh(**_MESH),
        scratch_types=scratch, compiler_params=_SC_PARAMS)


def _emb_body(x_ref, w_ref, b_ref, o_ref):
    o_ref[...] = jnp.dot(x_ref[...], w_ref[...],
                         preferred_element_type=jnp.float32,
                         precision=lax.Precision.HIGHEST) + b_ref[...]


def _embed(x, w, b, blk):
    n = x.shape[0]
    return pl.pallas_call(
        _emb_body,
        grid=(n // blk,),
        in_specs=[
            pl.BlockSpec((blk, H), lambda i: (i, 0)),
            pl.BlockSpec((H, H), lambda i: (0, 0)),
            pl.BlockSpec((1, H), lambda i: (0, 0)),
        ],
        out_specs=pl.BlockSpec((blk, H), lambda i: (i, 0)),
        out_shape=jax.ShapeDtypeStruct((n, H), jnp.float32),
    )(x, w, b.reshape(1, H))


def _layer_body(h_ref, s_ref, deg_ref, wt_ref, wb_ref, p_ref, o_ref):
    h = h_ref[...]
    s_sum = s_ref[0] + s_ref[1]
    deg = deg_ref[0, :, 0:1] + deg_ref[1, :, 0:1]
    b = p_ref[0:1, :]
    gamma = p_ref[1:2, :]
    beta = p_ref[2:3, :]
    mean = p_ref[3:4, :]
    var = p_ref[4:5, :]
    hw = jnp.dot(h, wb_ref[...], preferred_element_type=jnp.float32,
                 precision=lax.Precision.HIGHEST)
    agg = jnp.dot(s_sum, wt_ref[...], preferred_element_type=jnp.float32,
                  precision=lax.Precision.HIGHEST) + deg * (hw + b)
    z = jax.nn.sigmoid(agg) + jax.nn.softplus(h)
    z = (z - mean) / jnp.sqrt(var + BN_EPS) * gamma + beta
    o_ref[...] = jnp.maximum(z, 0.0)


def _layer_dense(h, s_partial, deg_partial, w, params, blk):
    n = h.shape[0]
    return pl.pallas_call(
        _layer_body,
        grid=(n // blk,),
        in_specs=[
            pl.BlockSpec((blk, H), lambda i: (i, 0)),
            pl.BlockSpec((NC, blk, H), lambda i: (0, i, 0)),
            pl.BlockSpec((NC, blk, L), lambda i: (0, i, 0)),
            pl.BlockSpec((H, H), lambda i: (0, 0)),
            pl.BlockSpec((H, H), lambda i: (0, 0)),
            pl.BlockSpec((5, H), lambda i: (0, 0)),
        ],
        out_specs=pl.BlockSpec((blk, H), lambda i: (i, 0)),
        out_shape=jax.ShapeDtypeStruct((n, H), jnp.float32),
    )(h, s_partial, deg_partial, w[:H], w[H:], params)


def kernel(node_feat, edge_index, W_emb, b_emb,
           w0, b0, gamma0, beta0, mean0, var0,
           w1, b1, gamma1, beta1, mean1, var1):
    n = node_feat.shape[0]
    e = edge_index.shape[1]
    n_pad = ((n + 1 + NS * 8 - 1) // (NS * 8)) * (NS * 8)
    p_tot = (e + NS * CHUNK - 1) // (NS * CHUNK)
    while p_tot % 2 or p_tot % 3 != 1:
        p_tot += 1
    k1 = max(5, int(p_tot * 0.125))
    k1 = (k1 // 3) * 3 + 2
    k0 = p_tot - k1
    totc = NS * p_tot
    e_pad = totc * CHUNK

    src = edge_index[0].astype(jnp.int32)
    dst = edge_index[1].astype(jnp.int32)
    src_i = jnp.zeros((e_pad,), jnp.int32).at[:e].set(src).reshape(totc, CHUNK)
    dst_i = jnp.full((e_pad,), n, jnp.int32).at[:e].set(dst).reshape(totc, CHUNK)

    zeros_s = jnp.zeros((n_pad, H), jnp.float32)
    zeros_d = jnp.zeros((n_pad, L), jnp.float32)
    ones = jnp.ones((CHUNK, L), jnp.float32)

    sc_seg = _make_sc_segment_sum(n_pad, k0, k1)
    sc_deg = _make_sc_degree(n_pad, totc // NW)

    blk = 2000
    h = _embed(node_feat, W_emb, b_emb, blk)
    deg_p = sc_deg(dst_i, zeros_d, ones)

    s_p = sc_seg(h, src_i, dst_i, zeros_s)
    params0 = jnp.stack([b0, gamma0, beta0, mean0, var0])
    h = _layer_dense(h, s_p, deg_p, w0, params0, blk)

    s_p = sc_seg(h, src_i, dst_i, zeros_s)
    params1 = jnp.stack([b1, gamma1, beta1, mean1, var1])
    h = _layer_dense(h, s_p, deg_p, w1, params1, blk)

    return h
